# Initial kernel scaffold; baseline (speedup 1.0000x reference)
import functools

import jax
import jax.numpy as jnp
from jax import lax
from jax.experimental import pallas as pl
from jax.experimental.pallas import tpu as pltpu
from jax.experimental.pallas import tpu_sc as plsc

N = 10000
NP = 10240
E = 320000
EP = 327680
ER = EP // 128
KE = ER // 32
KC = ER // 16
SROWS = NP // 16
BLK = 640
GRID = NP // BLK

_F32 = jnp.float32


def _sc_mesh():
    return plsc.VectorSubcoreMesh(
        core_axis_name="c", subcore_axis_name="s", num_cores=2, num_subcores=16)


def _make_deg():
    @functools.partial(
        pl.kernel,
        out_type=jax.ShapeDtypeStruct((2, NP, 16), _F32),
        mesh=_sc_mesh(),
        scratch_types=[
            pltpu.VMEM((KE, 128), jnp.int32),
            pltpu.VMEM((128, 16), _F32),
            pltpu.VMEM((128, 16), _F32),
            pltpu.VMEM_SHARED((NP, 16), _F32),
        ],
    )
    def deg_kernel(dst2d, ones16, zeros16, out, dstv, onesv, zb, acc):
        c = lax.axis_index("c")
        s = lax.axis_index("s")
        tile = c * 16 + s
        pltpu.sync_copy(dst2d.at[pl.ds(tile * KE, KE)], dstv)
        pltpu.sync_copy(ones16, onesv)
        pltpu.sync_copy(zeros16, zb)
        for i in range(SROWS // 128):
            pltpu.sync_copy(zb, acc.at[pl.ds(s * SROWS + i * 128, 128)])
        plsc.subcore_barrier()

        def body(j, carry):
            pltpu.sync_copy(onesv, acc.at[dstv.at[j]], add=True)
            return carry

        lax.fori_loop(0, KE, body, 0)
        plsc.subcore_barrier()
        for i in range(SROWS // 128):
            r = s * SROWS + i * 128
            pltpu.sync_copy(acc.at[pl.ds(r, 128)], zb)
            pltpu.sync_copy(zb, out.at[c, pl.ds(r, 128)])

    return deg_kernel


def _make_agg(K, edge_split):
    @functools.partial(
        pl.kernel,
        out_type=jax.ShapeDtypeStruct((2, NP, 128), _F32),
        mesh=_sc_mesh(),
        scratch_types=[
            pltpu.VMEM((K, 128), jnp.int32),
            pltpu.VMEM((K, 128), jnp.int32),
            pltpu.VMEM((128, 128), _F32),
            pltpu.VMEM((128, 128), _F32),
            pltpu.VMEM_SHARED((NP, 128), _F32),
            pltpu.SemaphoreType.DMA,
            pltpu.SemaphoreType.DMA,
        ],
    )
    def agg_kernel(table, srci, dsti, zeros, out, srcv, dstv, buf0, buf1,
                   acc, gs0, gs1):
        c = lax.axis_index("c")
        s = lax.axis_index("s")
        if edge_split:
            row0 = (c * 16 + s) * K
            pltpu.sync_copy(srci.at[pl.ds(row0, K)], srcv)
        else:
            row0 = s * K
            pltpu.sync_copy(srci.at[c, pl.ds(row0, K)], srcv)
        pltpu.sync_copy(dsti.at[pl.ds(row0, K)], dstv)
        pltpu.sync_copy(zeros, buf0)
        for i in range(SROWS // 128):
            pltpu.sync_copy(buf0, acc.at[pl.ds(s * SROWS + i * 128, 128)])
        plsc.subcore_barrier()

        bufs = (buf0, buf1)
        sems = (gs0, gs1)
        pltpu.async_copy(table.at[srcv.at[0]], buf0, gs0)

        def outer(g, carry):
            for b in (0, 1):
                j = 2 * g + b
                nb = 1 - b
                jn = jnp.minimum(j + 1, K - 1)
                pltpu.async_copy(table.at[srcv.at[jn]], bufs[nb], sems[nb])
                pltpu.make_async_copy(table.at[srcv.at[j]], bufs[b],
                                      sems[b]).wait()
                pltpu.sync_copy(bufs[b], acc.at[dstv.at[j]], add=True)
            return carry

        lax.fori_loop(0, K // 2, outer, 0)
        pltpu.make_async_copy(table.at[srcv.at[K - 1]], buf0, gs0).wait()
        plsc.subcore_barrier()
        for i in range(SROWS // 128):
            r = s * SROWS + i * 128
            pltpu.sync_copy(acc.at[pl.ds(r, 128)], buf0)
            pltpu.sync_copy(buf0, out.at[c, pl.ds(r, 128)])

    return agg_kernel


def _dot(a, b):
    return jnp.dot(a, b, precision=lax.Precision.HIGHEST,
                   preferred_element_type=_F32)


def _tc_prep(degp, x_pad):
    def body(deg_ref, x_ref, dis_ref, u0_ref):
        d = deg_ref[0] + deg_ref[1] + 1.0
        dis = lax.rsqrt(jnp.maximum(d, 1.0))
        dis_ref[...] = dis
        u0_ref[...] = dis * x_ref[...]

    return pl.pallas_call(
        body,
        grid=(GRID,),
        in_specs=[
            pl.BlockSpec((2, BLK, 1), lambda i: (0, i, 0)),
            pl.BlockSpec((BLK, 128), lambda i: (i, 0)),
        ],
        out_specs=[
            pl.BlockSpec((BLK, 1), lambda i: (i, 0)),
            pl.BlockSpec((BLK, 128), lambda i: (i, 0)),
        ],
        out_shape=[
            jax.ShapeDtypeStruct((NP, 1), _F32),
            jax.ShapeDtypeStruct((NP, 128), _F32),
        ],
    )(degp, x_pad)


def _tc_layer1(acc0, u0, dis, W1, b1):
    def body(acc_ref, u0_ref, dis_ref, w_ref, b_ref, out_ref):
        i = pl.program_id(0)
        dis = dis_ref[...]
        g0 = dis * (acc_ref[0] + acc_ref[1] + u0_ref[...])
        h = jnp.maximum(_dot(g0, w_ref[...]) + b_ref[...], 0.0)
        u1 = dis * h
        rows = i * BLK + lax.broadcasted_iota(jnp.int32, (BLK, 1), 0)
        u1 = jnp.where(rows < N, u1, 0.0)
        out_ref[0] = u1[:, :128]
        out_ref[1] = u1[:, 128:]

    return pl.pallas_call(
        body,
        grid=(GRID,),
        in_specs=[
            pl.BlockSpec((2, BLK, 128), lambda i: (0, i, 0)),
            pl.BlockSpec((BLK, 128), lambda i: (i, 0)),
            pl.BlockSpec((BLK, 1), lambda i: (i, 0)),
            pl.BlockSpec((128, 256), lambda i: (0, 0)),
            pl.BlockSpec((1, 256), lambda i: (0, 0)),
        ],
        out_specs=[pl.BlockSpec((2, BLK, 128), lambda i: (0, i, 0))],
        out_shape=[jax.ShapeDtypeStruct((2, NP, 128), _F32)],
    )(acc0, u0, dis, W1, b1)[0]


def _tc_layer2(acc1, u1cat, dis, W_mu, b_mu, W_ls, b_ls):
    def body(acc_ref, u1_ref, dis_ref, wmu_ref, bmu_ref, wls_ref, bls_ref,
             mu_ref, ls_ref):
        dis = dis_ref[...]
        g1a = dis * (acc_ref[0] + u1_ref[0])
        g1b = dis * (acc_ref[1] + u1_ref[1])
        mu_ref[...] = _dot(g1a, wmu_ref[0]) + _dot(g1b, wmu_ref[1]) + bmu_ref[...]
        ls_ref[...] = _dot(g1a, wls_ref[0]) + _dot(g1b, wls_ref[1]) + bls_ref[...]

    return pl.pallas_call(
        body,
        grid=(GRID,),
        in_specs=[
            pl.BlockSpec((2, BLK, 128), lambda i: (0, i, 0)),
            pl.BlockSpec((2, BLK, 128), lambda i: (0, i, 0)),
            pl.BlockSpec((BLK, 1), lambda i: (i, 0)),
            pl.BlockSpec((2, 128, 128), lambda i: (0, 0, 0)),
            pl.BlockSpec((1, 128), lambda i: (0, 0)),
            pl.BlockSpec((2, 128, 128), lambda i: (0, 0, 0)),
            pl.BlockSpec((1, 128), lambda i: (0, 0)),
        ],
        out_specs=[
            pl.BlockSpec((BLK, 128), lambda i: (i, 0)),
            pl.BlockSpec((BLK, 128), lambda i: (i, 0)),
        ],
        out_shape=[
            jax.ShapeDtypeStruct((NP, 128), _F32),
            jax.ShapeDtypeStruct((NP, 128), _F32),
        ],
    )(acc1, u1cat, dis, W_mu, b_mu, W_ls, b_ls)


def kernel(x, edge_index, W1, b1, W_mu, b_mu, W_logstd, b_logstd):
    ei = edge_index.astype(jnp.int32)
    pad = jnp.full((EP - E,), N, jnp.int32)
    src2d = jnp.concatenate([ei[0], pad]).reshape(ER, 128)
    dst2d = jnp.concatenate([ei[1], pad]).reshape(ER, 128)
    src2d_both = jnp.stack([src2d, src2d + NP])
    x_pad = jnp.pad(x, ((0, NP - N), (0, 0)))
    zeros128 = jnp.zeros((128, 128), _F32)
    ones16 = jnp.ones((128, 16), _F32)
    zeros16 = jnp.zeros((128, 16), _F32)

    degp = _make_deg()(dst2d, ones16, zeros16)
    dis, u0 = _tc_prep(degp[:, :, :1], x_pad)
    acc0 = _make_agg(KE, True)(u0, src2d, dst2d, zeros128)
    u1cat = _tc_layer1(acc0, u0, dis, W1, b1.reshape(1, -1))
    u1flat = u1cat.reshape(2 * NP, 128)
    acc1 = _make_agg(KC, False)(u1flat, src2d_both, dst2d, zeros128)
    mu, logstd = _tc_layer2(acc1, u1cat, dis,
                            W_mu.reshape(2, 128, 128), b_mu.reshape(1, -1),
                            W_logstd.reshape(2, 128, 128),
                            b_logstd.reshape(1, -1))
    return mu[:N], logstd[:N]


# speedup vs baseline: 11.0223x; 11.0223x over previous
import functools

import jax
import jax.numpy as jnp
from jax import lax
from jax.experimental import pallas as pl
from jax.experimental.pallas import tpu as pltpu
from jax.experimental.pallas import tpu_sc as plsc

N = 10000
NP = 10240
E = 320000
EP = 327680
ER = EP // 128
KE = ER // 32
KC = ER // 16
SROWS = NP // 16
BLK = 640
GRID = NP // BLK

_F32 = jnp.float32


def _sc_mesh():
    return plsc.VectorSubcoreMesh(
        core_axis_name="c", subcore_axis_name="s", num_cores=2, num_subcores=16)


def _make_deg():
    @functools.partial(
        pl.kernel,
        out_type=jax.ShapeDtypeStruct((2, NP, 128), _F32),
        mesh=_sc_mesh(),
        scratch_types=[
            pltpu.VMEM((KE, 128), jnp.int32),
            pltpu.VMEM((128, 128), _F32),
            pltpu.VMEM_SHARED((NP, 128), _F32),
            pltpu.SemaphoreType.DMA,
        ],
    )
    def deg_kernel(dst2d, ones128, zeros128, out, dstv, buf, acc, sem):
        c = lax.axis_index("c")
        s = lax.axis_index("s")
        tile = c * 16 + s
        pltpu.sync_copy(dst2d.at[pl.ds(tile * KE, KE)], dstv)
        pltpu.sync_copy(zeros128, buf)
        for i in range(SROWS // 128):
            pltpu.sync_copy(buf, acc.at[pl.ds(s * SROWS + i * 128, 128)])
        plsc.subcore_barrier()
        pltpu.sync_copy(ones128, buf)

        def body(g, carry):
            descs = []
            for b in range(4):
                j = 4 * g + b
                descs.append(pltpu.async_copy(buf, acc.at[dstv.at[j]], sem,
                                              add=True))
            for d in descs:
                d.wait()
            return carry

        lax.fori_loop(0, KE // 4, body, 0)
        plsc.subcore_barrier()
        for i in range(SROWS // 128):
            r = s * SROWS + i * 128
            pltpu.sync_copy(acc.at[pl.ds(r, 128)], buf)
            pltpu.sync_copy(buf, out.at[c, pl.ds(r, 128)])

    return deg_kernel


def _make_agg(K, RB, edge_split):
    assert K % RB == 0 and RB % 2 == 0
    CH = K // RB

    @functools.partial(
        pl.kernel,
        out_type=jax.ShapeDtypeStruct((2, NP, 128), _F32),
        mesh=_sc_mesh(),
        scratch_types=[
            pltpu.VMEM((RB, 128), jnp.int32),
            pltpu.VMEM((RB, 128), jnp.int32),
            pltpu.VMEM((128, 128), _F32),
            pltpu.VMEM((128, 128), _F32),
            pltpu.VMEM_SHARED((NP, 128), _F32),
            pltpu.SemaphoreType.DMA,
            pltpu.SemaphoreType.DMA,
        ],
    )
    def agg_kernel(table, srci, dsti, zeros, out, srcv, dstv, buf0, buf1,
                   acc, gs0, gs1):
        c = lax.axis_index("c")
        s = lax.axis_index("s")
        row0 = (c * 16 + s) * K if edge_split else s * K
        pltpu.sync_copy(zeros, buf0)
        for i in range(SROWS // 128):
            pltpu.sync_copy(buf0, acc.at[pl.ds(s * SROWS + i * 128, 128)])
        plsc.subcore_barrier()

        bufs = (buf0, buf1)
        sems = (gs0, gs1)

        def chunk(ci, carry):
            base = row0 + ci * RB
            if edge_split:
                pltpu.sync_copy(srci.at[pl.ds(base, RB)], srcv)
            else:
                pltpu.sync_copy(srci.at[c, pl.ds(base, RB)], srcv)
            pltpu.sync_copy(dsti.at[pl.ds(base, RB)], dstv)
            pltpu.async_copy(table.at[srcv.at[0]], buf0, gs0)

            def inner(g, carry2):
                for b in (0, 1):
                    j = 2 * g + b
                    nb = 1 - b
                    jn = jnp.minimum(j + 1, RB - 1)
                    pltpu.async_copy(table.at[srcv.at[jn]], bufs[nb], sems[nb])
                    pltpu.make_async_copy(table.at[srcv.at[j]], bufs[b],
                                          sems[b]).wait()
                    pltpu.sync_copy(bufs[b], acc.at[dstv.at[j]], add=True)
                return carry2

            lax.fori_loop(0, RB // 2, inner, 0)
            pltpu.make_async_copy(table.at[srcv.at[RB - 1]], buf0, gs0).wait()
            return carry

        lax.fori_loop(0, CH, chunk, 0)
        plsc.subcore_barrier()
        for i in range(SROWS // 128):
            r = s * SROWS + i * 128
            pltpu.sync_copy(acc.at[pl.ds(r, 128)], buf0)
            pltpu.sync_copy(buf0, out.at[c, pl.ds(r, 128)])

    return agg_kernel


def _dot(a, b):
    return jnp.dot(a, b, precision=lax.Precision.HIGHEST,
                   preferred_element_type=_F32)


def _tc_prep(degp, x_pad):
    def body(deg_ref, x_ref, dis_ref, u0_ref):
        d = deg_ref[0] + deg_ref[1] + 1.0
        dis = lax.rsqrt(jnp.maximum(d, 1.0))
        dis_ref[...] = dis
        u0_ref[...] = dis * x_ref[...]

    return pl.pallas_call(
        body,
        grid=(GRID,),
        in_specs=[
            pl.BlockSpec((2, BLK, 1), lambda i: (0, i, 0)),
            pl.BlockSpec((BLK, 128), lambda i: (i, 0)),
        ],
        out_specs=[
            pl.BlockSpec((BLK, 1), lambda i: (i, 0)),
            pl.BlockSpec((BLK, 128), lambda i: (i, 0)),
        ],
        out_shape=[
            jax.ShapeDtypeStruct((NP, 1), _F32),
            jax.ShapeDtypeStruct((NP, 128), _F32),
        ],
    )(degp, x_pad)


def _tc_layer1(acc0, u0, dis, W1, b1):
    def body(acc_ref, u0_ref, dis_ref, w_ref, b_ref, out_ref):
        i = pl.program_id(0)
        dis = dis_ref[...]
        g0 = dis * (acc_ref[0] + acc_ref[1] + u0_ref[...])
        h = jnp.maximum(_dot(g0, w_ref[...]) + b_ref[...], 0.0)
        u1 = dis * h
        rows = i * BLK + lax.broadcasted_iota(jnp.int32, (BLK, 1), 0)
        u1 = jnp.where(rows < N, u1, 0.0)
        out_ref[0] = u1[:, :128]
        out_ref[1] = u1[:, 128:]

    return pl.pallas_call(
        body,
        grid=(GRID,),
        in_specs=[
            pl.BlockSpec((2, BLK, 128), lambda i: (0, i, 0)),
            pl.BlockSpec((BLK, 128), lambda i: (i, 0)),
            pl.BlockSpec((BLK, 1), lambda i: (i, 0)),
            pl.BlockSpec((128, 256), lambda i: (0, 0)),
            pl.BlockSpec((1, 256), lambda i: (0, 0)),
        ],
        out_specs=[pl.BlockSpec((2, BLK, 128), lambda i: (0, i, 0))],
        out_shape=[jax.ShapeDtypeStruct((2, NP, 128), _F32)],
    )(acc0, u0, dis, W1, b1)[0]


def _tc_layer2(acc1, u1cat, dis, W_mu, b_mu, W_ls, b_ls):
    def body(acc_ref, u1_ref, dis_ref, wmu_ref, bmu_ref, wls_ref, bls_ref,
             mu_ref, ls_ref):
        dis = dis_ref[...]
        g1a = dis * (acc_ref[0] + u1_ref[0])
        g1b = dis * (acc_ref[1] + u1_ref[1])
        mu_ref[...] = _dot(g1a, wmu_ref[0]) + _dot(g1b, wmu_ref[1]) + bmu_ref[...]
        ls_ref[...] = _dot(g1a, wls_ref[0]) + _dot(g1b, wls_ref[1]) + bls_ref[...]

    return pl.pallas_call(
        body,
        grid=(GRID,),
        in_specs=[
            pl.BlockSpec((2, BLK, 128), lambda i: (0, i, 0)),
            pl.BlockSpec((2, BLK, 128), lambda i: (0, i, 0)),
            pl.BlockSpec((BLK, 1), lambda i: (i, 0)),
            pl.BlockSpec((2, 128, 128), lambda i: (0, 0, 0)),
            pl.BlockSpec((1, 128), lambda i: (0, 0)),
            pl.BlockSpec((2, 128, 128), lambda i: (0, 0, 0)),
            pl.BlockSpec((1, 128), lambda i: (0, 0)),
        ],
        out_specs=[
            pl.BlockSpec((BLK, 128), lambda i: (i, 0)),
            pl.BlockSpec((BLK, 128), lambda i: (i, 0)),
        ],
        out_shape=[
            jax.ShapeDtypeStruct((NP, 128), _F32),
            jax.ShapeDtypeStruct((NP, 128), _F32),
        ],
    )(acc1, u1cat, dis, W_mu, b_mu, W_ls, b_ls)


def kernel(x, edge_index, W1, b1, W_mu, b_mu, W_logstd, b_logstd):
    ei = edge_index.astype(jnp.int32)
    pad = jnp.full((EP - E,), N, jnp.int32)
    src2d = jnp.concatenate([ei[0], pad]).reshape(ER, 128)
    dst2d = jnp.concatenate([ei[1], pad]).reshape(ER, 128)
    src2d_both = jnp.stack([src2d, src2d + NP])
    x_pad = jnp.pad(x, ((0, NP - N), (0, 0)))
    zeros128 = jnp.zeros((128, 128), _F32)
    ones128 = jnp.ones((128, 128), _F32)

    degp = _make_deg()(dst2d, ones128, zeros128)
    dis, u0 = _tc_prep(degp[:, :, :1], x_pad)
    acc0 = _make_agg(KE, 16, True)(u0, src2d, dst2d, zeros128)
    u1cat = _tc_layer1(acc0, u0, dis, W1, b1.reshape(1, -1))
    u1flat = u1cat.reshape(2 * NP, 128)
    acc1 = _make_agg(KC, 32, False)(u1flat, src2d_both, dst2d, zeros128)
    mu, logstd = _tc_layer2(acc1, u1cat, dis,
                            W_mu.reshape(2, 128, 128), b_mu.reshape(1, -1),
                            W_logstd.reshape(2, 128, 128),
                            b_logstd.reshape(1, -1))
    return mu[:N], logstd[:N]


# speedup vs baseline: 25.7339x; 2.3347x over previous
import functools

import jax
import jax.numpy as jnp
from jax import lax
from jax.experimental import pallas as pl
from jax.experimental.pallas import tpu as pltpu
from jax.experimental.pallas import tpu_sc as plsc

N = 10000
NP = 10240
E = 320000
EP = 327680
ER = EP // 128
KE = ER // 32
KC = ER // 16
SROWS = NP // 16
BLK = 640
GRID = NP // BLK

_F32 = jnp.float32


def _sc_mesh():
    return plsc.VectorSubcoreMesh(
        core_axis_name="c", subcore_axis_name="s", num_cores=2, num_subcores=16)


def _make_deg():
    @functools.partial(
        pl.kernel,
        out_type=jax.ShapeDtypeStruct((2, NP, 128), _F32),
        mesh=_sc_mesh(),
        scratch_types=[
            pltpu.VMEM((KE, 128), jnp.int32),
            pltpu.VMEM((128, 128), _F32),
            pltpu.VMEM_SHARED((NP, 128), _F32),
            pltpu.SemaphoreType.DMA,
        ],
    )
    def deg_kernel(dst2d, ones128, zeros128, out, dstv, buf, acc, sem):
        c = lax.axis_index("c")
        s = lax.axis_index("s")
        tile = c * 16 + s
        pltpu.sync_copy(dst2d.at[pl.ds(tile * KE, KE)], dstv)
        pltpu.sync_copy(zeros128, buf)
        for i in range(SROWS // 128):
            pltpu.sync_copy(buf, acc.at[pl.ds(s * SROWS + i * 128, 128)])
        plsc.subcore_barrier()
        pltpu.sync_copy(ones128, buf)

        def body(g, carry):
            descs = []
            for b in range(4):
                j = 4 * g + b
                descs.append(pltpu.async_copy(buf, acc.at[dstv.at[j]], sem,
                                              add=True))
            for d in descs:
                d.wait()
            return carry

        lax.fori_loop(0, KE // 4, body, 0)
        plsc.subcore_barrier()
        for i in range(SROWS // 128):
            r = s * SROWS + i * 128
            pltpu.sync_copy(acc.at[pl.ds(r, 128)], buf)
            pltpu.sync_copy(buf, out.at[c, pl.ds(r, 128)])

    return deg_kernel


def _make_agg(K, RB, edge_split):
    assert K % RB == 0 and RB % 2 == 0
    CH = K // RB

    @functools.partial(
        pl.kernel,
        out_type=jax.ShapeDtypeStruct((2, NP, 128), _F32),
        mesh=_sc_mesh(),
        scratch_types=[
            pltpu.VMEM((RB, 128), jnp.int32),
            pltpu.VMEM((RB, 128), jnp.int32),
            pltpu.VMEM((128, 128), _F32),
            pltpu.VMEM((128, 128), _F32),
            pltpu.VMEM_SHARED((NP, 128), _F32),
            pltpu.SemaphoreType.DMA,
            pltpu.SemaphoreType.DMA,
        ],
    )
    def agg_kernel(table, srci, dsti, zeros, out, srcv, dstv, buf0, buf1,
                   acc, gs0, gs1):
        c = lax.axis_index("c")
        s = lax.axis_index("s")
        row0 = (c * 16 + s) * K if edge_split else s * K
        pltpu.sync_copy(zeros, buf0)
        for i in range(SROWS // 128):
            pltpu.sync_copy(buf0, acc.at[pl.ds(s * SROWS + i * 128, 128)])
        plsc.subcore_barrier()

        bufs = (buf0, buf1)
        sems = (gs0, gs1)

        def chunk(ci, carry):
            base = row0 + ci * RB
            if edge_split:
                pltpu.sync_copy(srci.at[pl.ds(base, RB)], srcv)
            else:
                pltpu.sync_copy(srci.at[c, pl.ds(base, RB)], srcv)
            pltpu.sync_copy(dsti.at[pl.ds(base, RB)], dstv)
            pltpu.async_copy(table.at[srcv.at[0]], buf0, gs0)

            def inner(g, carry2):
                for b in (0, 1):
                    j = 2 * g + b
                    nb = 1 - b
                    jn = jnp.minimum(j + 1, RB - 1)
                    pltpu.async_copy(table.at[srcv.at[jn]], bufs[nb], sems[nb])
                    pltpu.make_async_copy(table.at[srcv.at[j]], bufs[b],
                                          sems[b]).wait()
                    pltpu.sync_copy(bufs[b], acc.at[dstv.at[j]], add=True)
                return carry2

            lax.fori_loop(0, RB // 2, inner, 0)
            pltpu.make_async_copy(table.at[srcv.at[RB - 1]], buf0, gs0).wait()
            return carry

        lax.fori_loop(0, CH, chunk, 0)
        plsc.subcore_barrier()
        for i in range(SROWS // 128):
            r = s * SROWS + i * 128
            pltpu.sync_copy(acc.at[pl.ds(r, 128)], buf0)
            pltpu.sync_copy(buf0, out.at[c, pl.ds(r, 128)])

    return agg_kernel


def _dot(a, b):
    return jnp.dot(a, b, precision=lax.Precision.HIGHEST,
                   preferred_element_type=_F32)


def _tc_prep(degp, x_pad):
    def body(deg_ref, x_ref, dis_ref, u0_ref):
        d = deg_ref[0] + deg_ref[1] + 1.0
        dis = lax.rsqrt(jnp.maximum(d, 1.0))
        dis_ref[...] = dis
        u0_ref[...] = dis * x_ref[...]

    return pl.pallas_call(
        body,
        grid=(GRID,),
        in_specs=[
            pl.BlockSpec((2, BLK, 1), lambda i: (0, i, 0)),
            pl.BlockSpec((BLK, 128), lambda i: (i, 0)),
        ],
        out_specs=[
            pl.BlockSpec((BLK, 1), lambda i: (i, 0)),
            pl.BlockSpec((BLK, 128), lambda i: (i, 0)),
        ],
        out_shape=[
            jax.ShapeDtypeStruct((NP, 1), _F32),
            jax.ShapeDtypeStruct((NP, 128), _F32),
        ],
    )(degp, x_pad)


def _tc_layer1(acc0, u0, dis, W1, b1):
    def body(acc_ref, u0_ref, dis_ref, w_ref, b_ref, out_ref):
        i = pl.program_id(0)
        dis = dis_ref[...]
        g0 = dis * (acc_ref[0] + acc_ref[1] + u0_ref[...])
        h = jnp.maximum(_dot(g0, w_ref[...]) + b_ref[...], 0.0)
        u1 = dis * h
        rows = i * BLK + lax.broadcasted_iota(jnp.int32, (BLK, 1), 0)
        u1 = jnp.where(rows < N, u1, 0.0)
        out_ref[0] = u1[:, :128]
        out_ref[1] = u1[:, 128:]

    return pl.pallas_call(
        body,
        grid=(GRID,),
        in_specs=[
            pl.BlockSpec((2, BLK, 128), lambda i: (0, i, 0)),
            pl.BlockSpec((BLK, 128), lambda i: (i, 0)),
            pl.BlockSpec((BLK, 1), lambda i: (i, 0)),
            pl.BlockSpec((128, 256), lambda i: (0, 0)),
            pl.BlockSpec((1, 256), lambda i: (0, 0)),
        ],
        out_specs=[pl.BlockSpec((2, BLK, 128), lambda i: (0, i, 0))],
        out_shape=[jax.ShapeDtypeStruct((2, NP, 128), _F32)],
    )(acc0, u0, dis, W1, b1)[0]


def _tc_layer2(acc1, u1cat, dis, W_mu, b_mu, W_ls, b_ls):
    def body(acc_ref, u1_ref, dis_ref, wmu_ref, bmu_ref, wls_ref, bls_ref,
             mu_ref, ls_ref):
        dis = dis_ref[...]
        g1a = dis * (acc_ref[0] + u1_ref[0])
        g1b = dis * (acc_ref[1] + u1_ref[1])
        mu_ref[...] = _dot(g1a, wmu_ref[0]) + _dot(g1b, wmu_ref[1]) + bmu_ref[...]
        ls_ref[...] = _dot(g1a, wls_ref[0]) + _dot(g1b, wls_ref[1]) + bls_ref[...]

    return pl.pallas_call(
        body,
        grid=(GRID,),
        in_specs=[
            pl.BlockSpec((2, BLK, 128), lambda i: (0, i, 0)),
            pl.BlockSpec((2, BLK, 128), lambda i: (0, i, 0)),
            pl.BlockSpec((BLK, 1), lambda i: (i, 0)),
            pl.BlockSpec((2, 128, 128), lambda i: (0, 0, 0)),
            pl.BlockSpec((1, 128), lambda i: (0, 0)),
            pl.BlockSpec((2, 128, 128), lambda i: (0, 0, 0)),
            pl.BlockSpec((1, 128), lambda i: (0, 0)),
        ],
        out_specs=[
            pl.BlockSpec((BLK, 128), lambda i: (i, 0)),
            pl.BlockSpec((BLK, 128), lambda i: (i, 0)),
        ],
        out_shape=[
            jax.ShapeDtypeStruct((NP, 128), _F32),
            jax.ShapeDtypeStruct((NP, 128), _F32),
        ],
    )(acc1, u1cat, dis, W_mu, b_mu, W_ls, b_ls)


def kernel(x, edge_index, W1, b1, W_mu, b_mu, W_logstd, b_logstd):
    ei = edge_index.astype(jnp.int32)
    pad = N + (jnp.arange(EP - E, dtype=jnp.int32) % (NP - N))
    src2d = jnp.concatenate([ei[0], pad]).reshape(ER, 128)
    dst2d = jnp.concatenate([ei[1], pad]).reshape(ER, 128)
    src2d_both = jnp.stack([src2d, src2d + NP])
    x_pad = jnp.pad(x, ((0, NP - N), (0, 0)))
    zeros128 = jnp.zeros((128, 128), _F32)
    ones128 = jnp.ones((128, 128), _F32)

    degp = _make_deg()(dst2d, ones128, zeros128)
    dis, u0 = _tc_prep(degp[:, :, :1], x_pad)
    acc0 = _make_agg(KE, 16, True)(u0, src2d, dst2d, zeros128)
    u1cat = _tc_layer1(acc0, u0, dis, W1, b1.reshape(1, -1))
    u1flat = u1cat.reshape(2 * NP, 128)
    acc1 = _make_agg(KC, 32, False)(u1flat, src2d_both, dst2d, zeros128)
    mu, logstd = _tc_layer2(acc1, u1cat, dis,
                            W_mu.reshape(2, 128, 128), b_mu.reshape(1, -1),
                            W_logstd.reshape(2, 128, 128),
                            b_logstd.reshape(1, -1))
    return mu[:N], logstd[:N]


# speedup vs baseline: 26.0572x; 1.0126x over previous
import functools

import jax
import jax.numpy as jnp
from jax import lax
from jax.experimental import pallas as pl
from jax.experimental.pallas import tpu as pltpu
from jax.experimental.pallas import tpu_sc as plsc

N = 10000
NP = 10240
E = 320000
EP = 327680
ER = EP // 128
KE = ER // 32
KC = ER // 16
SROWS = NP // 16
BLK = 640
GRID = NP // BLK

_F32 = jnp.float32


def _sc_mesh():
    return plsc.VectorSubcoreMesh(
        core_axis_name="c", subcore_axis_name="s", num_cores=2, num_subcores=16)


def _make_deg():
    @functools.partial(
        pl.kernel,
        out_type=jax.ShapeDtypeStruct((2, NP, 128), _F32),
        mesh=_sc_mesh(),
        scratch_types=[
            pltpu.VMEM((KE, 128), jnp.int32),
            pltpu.VMEM((128, 128), _F32),
            pltpu.VMEM_SHARED((NP, 128), _F32),
        ],
    )
    def deg_kernel(dsti, zeros, ones, out, dstv, buf, acc):
        c = lax.axis_index("c")
        s = lax.axis_index("s")
        row0 = (c * 16 + s) * KE
        pltpu.sync_copy(zeros, buf)
        for i in range(SROWS // 128):
            pltpu.sync_copy(buf, acc.at[pl.ds(s * SROWS + i * 128, 128)])
        plsc.subcore_barrier()
        pltpu.sync_copy(dsti.at[pl.ds(row0, KE)], dstv)
        pltpu.sync_copy(ones, buf)

        def body(j, carry):
            pltpu.sync_copy(buf, acc.at[dstv.at[j]], add=True)
            return carry

        lax.fori_loop(0, KE, body, 0)
        plsc.subcore_barrier()
        for i in range(SROWS // 128):
            r = s * SROWS + i * 128
            pltpu.sync_copy(acc.at[pl.ds(r, 128)], buf)
            pltpu.sync_copy(buf, out.at[c, pl.ds(r, 128)])

    return deg_kernel


def _make_agg(K, RB, edge_split):
    assert K % RB == 0 and RB % 2 == 0
    CH = K // RB

    @functools.partial(
        pl.kernel,
        out_type=jax.ShapeDtypeStruct((2, NP, 128), _F32),
        mesh=_sc_mesh(),
        scratch_types=[
            pltpu.VMEM((RB, 128), jnp.int32),
            pltpu.VMEM((RB, 128), jnp.int32),
            pltpu.VMEM((128, 128), _F32),
            pltpu.VMEM((128, 128), _F32),
            pltpu.VMEM_SHARED((NP, 128), _F32),
            pltpu.SemaphoreType.DMA,
            pltpu.SemaphoreType.DMA,
        ],
    )
    def agg_kernel(table, srci, dsti, zeros, out, srcv, dstv, buf0, buf1,
                   acc, gs0, gs1):
        c = lax.axis_index("c")
        s = lax.axis_index("s")
        row0 = (c * 16 + s) * K if edge_split else s * K
        pltpu.sync_copy(zeros, buf0)
        for i in range(SROWS // 128):
            pltpu.sync_copy(buf0, acc.at[pl.ds(s * SROWS + i * 128, 128)])
        plsc.subcore_barrier()

        bufs = (buf0, buf1)
        sems = (gs0, gs1)

        def chunk(ci, carry):
            base = row0 + ci * RB
            if edge_split:
                pltpu.sync_copy(srci.at[pl.ds(base, RB)], srcv)
            else:
                pltpu.sync_copy(srci.at[c, pl.ds(base, RB)], srcv)
            pltpu.sync_copy(dsti.at[pl.ds(base, RB)], dstv)
            pltpu.async_copy(table.at[srcv.at[0]], buf0, gs0)

            def inner(g, carry2):
                for b in (0, 1):
                    j = 2 * g + b
                    nb = 1 - b
                    jn = jnp.minimum(j + 1, RB - 1)
                    pltpu.async_copy(table.at[srcv.at[jn]], bufs[nb], sems[nb])
                    pltpu.make_async_copy(table.at[srcv.at[j]], bufs[b],
                                          sems[b]).wait()
                    pltpu.sync_copy(bufs[b], acc.at[dstv.at[j]], add=True)
                return carry2

            lax.fori_loop(0, RB // 2, inner, 0)
            pltpu.make_async_copy(table.at[srcv.at[RB - 1]], buf0, gs0).wait()
            return carry

        lax.fori_loop(0, CH, chunk, 0)
        plsc.subcore_barrier()
        for i in range(SROWS // 128):
            r = s * SROWS + i * 128
            pltpu.sync_copy(acc.at[pl.ds(r, 128)], buf0)
            pltpu.sync_copy(buf0, out.at[c, pl.ds(r, 128)])

    return agg_kernel


def _dot(a, b):
    return jnp.dot(a, b, precision=lax.Precision.DEFAULT,
                   preferred_element_type=_F32)


def _tc_prep(degp, x_pad):
    def body(deg_ref, x_ref, dis_ref, u0_ref):
        d = deg_ref[0][:, :1] + deg_ref[1][:, :1] + 1.0
        dis = lax.rsqrt(jnp.maximum(d, 1.0))
        dis_ref[...] = dis
        u0_ref[...] = dis * x_ref[...]

    return pl.pallas_call(
        body,
        grid=(GRID,),
        in_specs=[
            pl.BlockSpec((2, BLK, 128), lambda i: (0, i, 0)),
            pl.BlockSpec((BLK, 128), lambda i: (i, 0)),
        ],
        out_specs=[
            pl.BlockSpec((BLK, 1), lambda i: (i, 0)),
            pl.BlockSpec((BLK, 128), lambda i: (i, 0)),
        ],
        out_shape=[
            jax.ShapeDtypeStruct((NP, 1), _F32),
            jax.ShapeDtypeStruct((NP, 128), _F32),
        ],
    )(degp, x_pad)


def _tc_layer1(acc0, u0, dis, W1, b1):
    def body(acc_ref, u0_ref, dis_ref, w_ref, b_ref, out_ref):
        i = pl.program_id(0)
        dis = dis_ref[...]
        g0 = dis * (acc_ref[0] + acc_ref[1] + u0_ref[...])
        h = jnp.maximum(_dot(g0, w_ref[...]) + b_ref[...], 0.0)
        u1 = dis * h
        rows = i * BLK + lax.broadcasted_iota(jnp.int32, (BLK, 1), 0)
        u1 = jnp.where(rows < N, u1, 0.0)
        out_ref[0] = u1[:, :128]
        out_ref[1] = u1[:, 128:]

    return pl.pallas_call(
        body,
        grid=(GRID,),
        in_specs=[
            pl.BlockSpec((2, BLK, 128), lambda i: (0, i, 0)),
            pl.BlockSpec((BLK, 128), lambda i: (i, 0)),
            pl.BlockSpec((BLK, 1), lambda i: (i, 0)),
            pl.BlockSpec((128, 256), lambda i: (0, 0)),
            pl.BlockSpec((1, 256), lambda i: (0, 0)),
        ],
        out_specs=[pl.BlockSpec((2, BLK, 128), lambda i: (0, i, 0))],
        out_shape=[jax.ShapeDtypeStruct((2, NP, 128), _F32)],
    )(acc0, u0, dis, W1, b1)[0]


def _tc_layer2(acc1, u1cat, dis, W_mu, b_mu, W_ls, b_ls):
    def body(acc_ref, u1_ref, dis_ref, wmu_ref, bmu_ref, wls_ref, bls_ref,
             mu_ref, ls_ref):
        dis = dis_ref[...]
        g1a = dis * (acc_ref[0] + u1_ref[0])
        g1b = dis * (acc_ref[1] + u1_ref[1])
        mu_ref[...] = _dot(g1a, wmu_ref[0]) + _dot(g1b, wmu_ref[1]) + bmu_ref[...]
        ls_ref[...] = _dot(g1a, wls_ref[0]) + _dot(g1b, wls_ref[1]) + bls_ref[...]

    return pl.pallas_call(
        body,
        grid=(GRID,),
        in_specs=[
            pl.BlockSpec((2, BLK, 128), lambda i: (0, i, 0)),
            pl.BlockSpec((2, BLK, 128), lambda i: (0, i, 0)),
            pl.BlockSpec((BLK, 1), lambda i: (i, 0)),
            pl.BlockSpec((2, 128, 128), lambda i: (0, 0, 0)),
            pl.BlockSpec((1, 128), lambda i: (0, 0)),
            pl.BlockSpec((2, 128, 128), lambda i: (0, 0, 0)),
            pl.BlockSpec((1, 128), lambda i: (0, 0)),
        ],
        out_specs=[
            pl.BlockSpec((BLK, 128), lambda i: (i, 0)),
            pl.BlockSpec((BLK, 128), lambda i: (i, 0)),
        ],
        out_shape=[
            jax.ShapeDtypeStruct((NP, 128), _F32),
            jax.ShapeDtypeStruct((NP, 128), _F32),
        ],
    )(acc1, u1cat, dis, W_mu, b_mu, W_ls, b_ls)


def kernel(x, edge_index, W1, b1, W_mu, b_mu, W_logstd, b_logstd):
    ei = edge_index.astype(jnp.int32)
    pad = N + (jnp.arange(EP - E, dtype=jnp.int32) % (NP - N))
    src2d = jnp.concatenate([ei[0], pad]).reshape(ER, 128)
    dst2d = jnp.concatenate([ei[1], pad]).reshape(ER, 128)
    src2d_both = jnp.stack([src2d, src2d + NP])
    x_pad = jnp.pad(x, ((0, NP - N), (0, 0)))
    zeros128 = jnp.zeros((128, 128), _F32)
    ones128 = jnp.ones((128, 128), _F32)

    degp = _make_deg()(dst2d, zeros128, ones128)
    dis, u0 = _tc_prep(degp, x_pad)
    acc0 = _make_agg(KE, 16, True)(u0, src2d, dst2d, zeros128)
    u1cat = _tc_layer1(acc0, u0, dis, W1, b1.reshape(1, -1))
    u1flat = u1cat.reshape(2 * NP, 128)
    acc1 = _make_agg(KC, 32, False)(u1flat, src2d_both, dst2d, zeros128)
    mu, logstd = _tc_layer2(acc1, u1cat, dis,
                            W_mu.reshape(2, 128, 128), b_mu.reshape(1, -1),
                            W_logstd.reshape(2, 128, 128),
                            b_logstd.reshape(1, -1))
    return mu[:N], logstd[:N]


# speedup vs baseline: 26.1325x; 1.0029x over previous
import functools

import jax
import jax.numpy as jnp
from jax import lax
from jax.experimental import pallas as pl
from jax.experimental.pallas import tpu as pltpu
from jax.experimental.pallas import tpu_sc as plsc

N = 10000
NP = 10240
E = 320000
EP = 327680
ER = EP // 128
KE = ER // 32
KC = ER // 16
SROWS = NP // 16
BLK = 640
GRID = NP // BLK

_F32 = jnp.float32


def _sc_mesh():
    return plsc.VectorSubcoreMesh(
        core_axis_name="c", subcore_axis_name="s", num_cores=2, num_subcores=16)


def _make_deg():
    @functools.partial(
        pl.kernel,
        out_type=jax.ShapeDtypeStruct((2, NP, 128), _F32),
        mesh=_sc_mesh(),
        scratch_types=[
            pltpu.VMEM((KE, 128), jnp.int32),
            pltpu.VMEM((128, 128), _F32),
            pltpu.VMEM_SHARED((NP, 128), _F32),
        ],
    )
    def deg_kernel(dsti, zeros, ones, out, dstv, buf, acc):
        c = lax.axis_index("c")
        s = lax.axis_index("s")
        row0 = (c * 16 + s) * KE
        pltpu.sync_copy(zeros, buf)
        for i in range(SROWS // 128):
            pltpu.sync_copy(buf, acc.at[pl.ds(s * SROWS + i * 128, 128)])
        plsc.subcore_barrier()
        pltpu.sync_copy(dsti.at[pl.ds(row0, KE)], dstv)
        pltpu.sync_copy(ones, buf)

        def body(j, carry):
            pltpu.sync_copy(buf, acc.at[dstv.at[j]], add=True)
            return carry

        lax.fori_loop(0, KE, body, 0)
        plsc.subcore_barrier()
        pltpu.sync_copy(acc.at[pl.ds(s * SROWS, SROWS)],
                        out.at[c, pl.ds(s * SROWS, SROWS)])

    return deg_kernel


def _make_agg(K, RB, edge_split):
    assert K % RB == 0 and RB % 2 == 0
    CH = K // RB

    @functools.partial(
        pl.kernel,
        out_type=jax.ShapeDtypeStruct((2, NP, 128), _F32),
        mesh=_sc_mesh(),
        scratch_types=[
            pltpu.VMEM((RB, 128), jnp.int32),
            pltpu.VMEM((RB, 128), jnp.int32),
            pltpu.VMEM((128, 128), _F32),
            pltpu.VMEM((128, 128), _F32),
            pltpu.VMEM_SHARED((NP, 128), _F32),
            pltpu.SemaphoreType.DMA,
            pltpu.SemaphoreType.DMA,
        ],
    )
    def agg_kernel(table, srci, dsti, zeros, out, srcv, dstv, buf0, buf1,
                   acc, gs0, gs1):
        c = lax.axis_index("c")
        s = lax.axis_index("s")
        row0 = (c * 16 + s) * K if edge_split else s * K
        pltpu.sync_copy(zeros, buf0)
        for i in range(SROWS // 128):
            pltpu.sync_copy(buf0, acc.at[pl.ds(s * SROWS + i * 128, 128)])
        plsc.subcore_barrier()

        bufs = (buf0, buf1)
        sems = (gs0, gs1)

        def chunk(ci, carry):
            base = row0 + ci * RB
            if edge_split:
                pltpu.sync_copy(srci.at[pl.ds(base, RB)], srcv)
            else:
                pltpu.sync_copy(srci.at[c, pl.ds(base, RB)], srcv)
            pltpu.sync_copy(dsti.at[pl.ds(base, RB)], dstv)
            pltpu.async_copy(table.at[srcv.at[0]], buf0, gs0)

            def inner(g, carry2):
                for b in (0, 1):
                    j = 2 * g + b
                    nb = 1 - b
                    jn = jnp.minimum(j + 1, RB - 1)
                    pltpu.async_copy(table.at[srcv.at[jn]], bufs[nb], sems[nb])
                    pltpu.make_async_copy(table.at[srcv.at[j]], bufs[b],
                                          sems[b]).wait()
                    pltpu.sync_copy(bufs[b], acc.at[dstv.at[j]], add=True)
                return carry2

            lax.fori_loop(0, RB // 2, inner, 0)
            pltpu.make_async_copy(table.at[srcv.at[RB - 1]], buf0, gs0).wait()
            return carry

        lax.fori_loop(0, CH, chunk, 0)
        plsc.subcore_barrier()
        pltpu.sync_copy(acc.at[pl.ds(s * SROWS, SROWS)],
                        out.at[c, pl.ds(s * SROWS, SROWS)])

    return agg_kernel


def _dot(a, b):
    return jnp.dot(a, b, precision=lax.Precision.DEFAULT,
                   preferred_element_type=_F32)


def _tc_prep(degp, x_pad):
    def body(deg_ref, x_ref, dis_ref, u0_ref):
        d = deg_ref[0][:, :1] + deg_ref[1][:, :1] + 1.0
        dis = lax.rsqrt(jnp.maximum(d, 1.0))
        dis_ref[...] = dis
        u0_ref[...] = dis * x_ref[...]

    return pl.pallas_call(
        body,
        grid=(GRID,),
        in_specs=[
            pl.BlockSpec((2, BLK, 128), lambda i: (0, i, 0)),
            pl.BlockSpec((BLK, 128), lambda i: (i, 0)),
        ],
        out_specs=[
            pl.BlockSpec((BLK, 1), lambda i: (i, 0)),
            pl.BlockSpec((BLK, 128), lambda i: (i, 0)),
        ],
        out_shape=[
            jax.ShapeDtypeStruct((NP, 1), _F32),
            jax.ShapeDtypeStruct((NP, 128), _F32),
        ],
    )(degp, x_pad)


def _tc_layer1(acc0, u0, dis, W1, b1):
    def body(acc_ref, u0_ref, dis_ref, w_ref, b_ref, out_ref):
        i = pl.program_id(0)
        dis = dis_ref[...]
        g0 = dis * (acc_ref[0] + acc_ref[1] + u0_ref[...])
        h = jnp.maximum(_dot(g0, w_ref[...]) + b_ref[...], 0.0)
        u1 = dis * h
        rows = i * BLK + lax.broadcasted_iota(jnp.int32, (BLK, 1), 0)
        u1 = jnp.where(rows < N, u1, 0.0)
        out_ref[0] = u1[:, :128]
        out_ref[1] = u1[:, 128:]

    return pl.pallas_call(
        body,
        grid=(GRID,),
        in_specs=[
            pl.BlockSpec((2, BLK, 128), lambda i: (0, i, 0)),
            pl.BlockSpec((BLK, 128), lambda i: (i, 0)),
            pl.BlockSpec((BLK, 1), lambda i: (i, 0)),
            pl.BlockSpec((128, 256), lambda i: (0, 0)),
            pl.BlockSpec((1, 256), lambda i: (0, 0)),
        ],
        out_specs=[pl.BlockSpec((2, BLK, 128), lambda i: (0, i, 0))],
        out_shape=[jax.ShapeDtypeStruct((2, NP, 128), _F32)],
    )(acc0, u0, dis, W1, b1)[0]


def _tc_layer2(acc1, u1cat, dis, W_mu, b_mu, W_ls, b_ls):
    def body(acc_ref, u1_ref, dis_ref, wmu_ref, bmu_ref, wls_ref, bls_ref,
             mu_ref, ls_ref):
        dis = dis_ref[...]
        g1a = dis * (acc_ref[0] + u1_ref[0])
        g1b = dis * (acc_ref[1] + u1_ref[1])
        mu_ref[...] = _dot(g1a, wmu_ref[0]) + _dot(g1b, wmu_ref[1]) + bmu_ref[...]
        ls_ref[...] = _dot(g1a, wls_ref[0]) + _dot(g1b, wls_ref[1]) + bls_ref[...]

    return pl.pallas_call(
        body,
        grid=(GRID,),
        in_specs=[
            pl.BlockSpec((2, BLK, 128), lambda i: (0, i, 0)),
            pl.BlockSpec((2, BLK, 128), lambda i: (0, i, 0)),
            pl.BlockSpec((BLK, 1), lambda i: (i, 0)),
            pl.BlockSpec((2, 128, 128), lambda i: (0, 0, 0)),
            pl.BlockSpec((1, 128), lambda i: (0, 0)),
            pl.BlockSpec((2, 128, 128), lambda i: (0, 0, 0)),
            pl.BlockSpec((1, 128), lambda i: (0, 0)),
        ],
        out_specs=[
            pl.BlockSpec((BLK, 128), lambda i: (i, 0)),
            pl.BlockSpec((BLK, 128), lambda i: (i, 0)),
        ],
        out_shape=[
            jax.ShapeDtypeStruct((NP, 128), _F32),
            jax.ShapeDtypeStruct((NP, 128), _F32),
        ],
    )(acc1, u1cat, dis, W_mu, b_mu, W_ls, b_ls)


def kernel(x, edge_index, W1, b1, W_mu, b_mu, W_logstd, b_logstd):
    ei = edge_index.astype(jnp.int32)
    pad = N + (jnp.arange(EP - E, dtype=jnp.int32) % (NP - N))
    src2d = jnp.concatenate([ei[0], pad]).reshape(ER, 128)
    dst2d = jnp.concatenate([ei[1], pad]).reshape(ER, 128)
    src2d_both = jnp.stack([src2d, src2d + NP])
    x_pad = jnp.pad(x, ((0, NP - N), (0, 0)))
    zeros128 = jnp.zeros((128, 128), _F32)
    ones128 = jnp.ones((128, 128), _F32)

    degp = _make_deg()(dst2d, zeros128, ones128)
    dis, u0 = _tc_prep(degp, x_pad)
    acc0 = _make_agg(KE, 16, True)(u0, src2d, dst2d, zeros128)
    u1cat = _tc_layer1(acc0, u0, dis, W1, b1.reshape(1, -1))
    u1flat = u1cat.reshape(2 * NP, 128)
    acc1 = _make_agg(KC, 32, False)(u1flat, src2d_both, dst2d, zeros128)
    mu, logstd = _tc_layer2(acc1, u1cat, dis,
                            W_mu.reshape(2, 128, 128), b_mu.reshape(1, -1),
                            W_logstd.reshape(2, 128, 128),
                            b_logstd.reshape(1, -1))
    return mu[:N], logstd[:N]


# speedup vs baseline: 26.4776x; 1.0132x over previous
import functools

import jax
import jax.numpy as jnp
from jax import lax
from jax.experimental import pallas as pl
from jax.experimental.pallas import tpu as pltpu
from jax.experimental.pallas import tpu_sc as plsc

N = 10000
NP = 10240
E = 320000
EP = 327680
ER = EP // 128
KE = ER // 32
KC = ER // 16
SROWS = NP // 16
BLK = 640
GRID = NP // BLK

_F32 = jnp.float32


def _sc_mesh():
    return plsc.VectorSubcoreMesh(
        core_axis_name="c", subcore_axis_name="s", num_cores=2, num_subcores=16)


def _make_deg():
    @functools.partial(
        pl.kernel,
        out_type=jax.ShapeDtypeStruct((2, NP, 128), _F32),
        mesh=_sc_mesh(),
        scratch_types=[
            pltpu.VMEM((KE, 128), jnp.int32),
            pltpu.VMEM((128, 128), _F32),
            pltpu.VMEM_SHARED((NP, 128), _F32),
        ],
    )
    def deg_kernel(dsti, zeros, ones, out, dstv, buf, acc):
        c = lax.axis_index("c")
        s = lax.axis_index("s")
        row0 = (c * 16 + s) * KE
        pltpu.sync_copy(zeros, buf)
        for i in range(SROWS // 128):
            pltpu.sync_copy(buf, acc.at[pl.ds(s * SROWS + i * 128, 128)])
        plsc.subcore_barrier()
        pltpu.sync_copy(dsti.at[pl.ds(row0, KE)], dstv)
        pltpu.sync_copy(ones, buf)

        def body(j, carry):
            pltpu.sync_copy(buf, acc.at[dstv.at[j]], add=True)
            return carry

        lax.fori_loop(0, KE, body, 0)
        plsc.subcore_barrier()
        pltpu.sync_copy(acc.at[pl.ds(s * SROWS, SROWS)],
                        out.at[c, pl.ds(s * SROWS, SROWS)])

    return deg_kernel


def _make_agg(K, RB, edge_split):
    assert K % RB == 0 and RB % 2 == 0
    CH = K // RB

    @functools.partial(
        pl.kernel,
        out_type=jax.ShapeDtypeStruct((2, NP, 128), _F32),
        mesh=_sc_mesh(),
        scratch_types=[
            pltpu.VMEM((RB, 128), jnp.int32),
            pltpu.VMEM((RB, 128), jnp.int32),
            pltpu.VMEM((128, 128), _F32),
            pltpu.VMEM((128, 128), _F32),
            pltpu.VMEM_SHARED((NP, 128), _F32),
            pltpu.SemaphoreType.DMA,
            pltpu.SemaphoreType.DMA,
        ],
    )
    def agg_kernel(table, srci, dsti, zeros, out, srcv, dstv, buf0, buf1,
                   acc, gs0, gs1):
        c = lax.axis_index("c")
        s = lax.axis_index("s")
        row0 = (c * 16 + s) * K if edge_split else s * K
        pltpu.sync_copy(zeros, buf0)
        for i in range(SROWS // 128):
            pltpu.sync_copy(buf0, acc.at[pl.ds(s * SROWS + i * 128, 128)])
        plsc.subcore_barrier()

        bufs = (buf0, buf1)
        sems = (gs0, gs1)

        def chunk(ci, carry):
            base = row0 + ci * RB
            if edge_split:
                pltpu.sync_copy(srci.at[pl.ds(base, RB)], srcv)
            else:
                pltpu.sync_copy(srci.at[c, pl.ds(base, RB)], srcv)
            pltpu.sync_copy(dsti.at[pl.ds(base, RB)], dstv)
            pltpu.async_copy(table.at[srcv.at[0]], buf0, gs0)

            def inner(g, carry2):
                for b in (0, 1):
                    j = 2 * g + b
                    nb = 1 - b

                    @pl.when(j + 1 < RB)
                    def _():
                        pltpu.async_copy(table.at[srcv.at[j + 1]], bufs[nb],
                                         sems[nb])

                    pltpu.make_async_copy(table.at[srcv.at[j]], bufs[b],
                                          sems[b]).wait()
                    pltpu.sync_copy(bufs[b], acc.at[dstv.at[j]], add=True)
                return carry2

            lax.fori_loop(0, RB // 2, inner, 0)
            return carry

        lax.fori_loop(0, CH, chunk, 0)
        plsc.subcore_barrier()
        pltpu.sync_copy(acc.at[pl.ds(s * SROWS, SROWS)],
                        out.at[c, pl.ds(s * SROWS, SROWS)])

    return agg_kernel


def _dot(a, b):
    return jnp.dot(a, b, precision=lax.Precision.DEFAULT,
                   preferred_element_type=_F32)


def _tc_prep(degp, x_pad):
    def body(deg_ref, x_ref, dis_ref, u0_ref):
        d = deg_ref[0][:, :1] + deg_ref[1][:, :1] + 1.0
        dis = lax.rsqrt(jnp.maximum(d, 1.0))
        dis_ref[...] = dis
        u0_ref[...] = dis * x_ref[...]

    return pl.pallas_call(
        body,
        grid=(GRID,),
        in_specs=[
            pl.BlockSpec((2, BLK, 128), lambda i: (0, i, 0)),
            pl.BlockSpec((BLK, 128), lambda i: (i, 0)),
        ],
        out_specs=[
            pl.BlockSpec((BLK, 1), lambda i: (i, 0)),
            pl.BlockSpec((BLK, 128), lambda i: (i, 0)),
        ],
        out_shape=[
            jax.ShapeDtypeStruct((NP, 1), _F32),
            jax.ShapeDtypeStruct((NP, 128), _F32),
        ],
    )(degp, x_pad)


def _tc_layer1(acc0, u0, dis, W1, b1):
    def body(acc_ref, u0_ref, dis_ref, w_ref, b_ref, out_ref):
        i = pl.program_id(0)
        dis = dis_ref[...]
        g0 = dis * (acc_ref[0] + acc_ref[1] + u0_ref[...])
        h = jnp.maximum(_dot(g0, w_ref[...]) + b_ref[...], 0.0)
        u1 = dis * h
        rows = i * BLK + lax.broadcasted_iota(jnp.int32, (BLK, 1), 0)
        u1 = jnp.where(rows < N, u1, 0.0)
        out_ref[0] = u1[:, :128]
        out_ref[1] = u1[:, 128:]

    return pl.pallas_call(
        body,
        grid=(GRID,),
        in_specs=[
            pl.BlockSpec((2, BLK, 128), lambda i: (0, i, 0)),
            pl.BlockSpec((BLK, 128), lambda i: (i, 0)),
            pl.BlockSpec((BLK, 1), lambda i: (i, 0)),
            pl.BlockSpec((128, 256), lambda i: (0, 0)),
            pl.BlockSpec((1, 256), lambda i: (0, 0)),
        ],
        out_specs=[pl.BlockSpec((2, BLK, 128), lambda i: (0, i, 0))],
        out_shape=[jax.ShapeDtypeStruct((2, NP, 128), _F32)],
    )(acc0, u0, dis, W1, b1)[0]


def _tc_layer2(acc1, u1cat, dis, W_mu, b_mu, W_ls, b_ls):
    def body(acc_ref, u1_ref, dis_ref, wmu_ref, bmu_ref, wls_ref, bls_ref,
             mu_ref, ls_ref):
        dis = dis_ref[...]
        g1a = dis * (acc_ref[0] + u1_ref[0])
        g1b = dis * (acc_ref[1] + u1_ref[1])
        mu_ref[...] = _dot(g1a, wmu_ref[0]) + _dot(g1b, wmu_ref[1]) + bmu_ref[...]
        ls_ref[...] = _dot(g1a, wls_ref[0]) + _dot(g1b, wls_ref[1]) + bls_ref[...]

    return pl.pallas_call(
        body,
        grid=(GRID,),
        in_specs=[
            pl.BlockSpec((2, BLK, 128), lambda i: (0, i, 0)),
            pl.BlockSpec((2, BLK, 128), lambda i: (0, i, 0)),
            pl.BlockSpec((BLK, 1), lambda i: (i, 0)),
            pl.BlockSpec((2, 128, 128), lambda i: (0, 0, 0)),
            pl.BlockSpec((1, 128), lambda i: (0, 0)),
            pl.BlockSpec((2, 128, 128), lambda i: (0, 0, 0)),
            pl.BlockSpec((1, 128), lambda i: (0, 0)),
        ],
        out_specs=[
            pl.BlockSpec((BLK, 128), lambda i: (i, 0)),
            pl.BlockSpec((BLK, 128), lambda i: (i, 0)),
        ],
        out_shape=[
            jax.ShapeDtypeStruct((NP, 128), _F32),
            jax.ShapeDtypeStruct((NP, 128), _F32),
        ],
    )(acc1, u1cat, dis, W_mu, b_mu, W_ls, b_ls)


def kernel(x, edge_index, W1, b1, W_mu, b_mu, W_logstd, b_logstd):
    ei = edge_index.astype(jnp.int32)
    pad = N + (jnp.arange(EP - E, dtype=jnp.int32) % (NP - N))
    src2d = jnp.concatenate([ei[0], pad]).reshape(ER, 128)
    dst2d = jnp.concatenate([ei[1], pad]).reshape(ER, 128)
    src2d_both = jnp.stack([src2d, src2d + NP])
    x_pad = jnp.pad(x, ((0, NP - N), (0, 0)))
    zeros128 = jnp.zeros((128, 128), _F32)
    ones128 = jnp.ones((128, 128), _F32)

    degp = _make_deg()(dst2d, zeros128, ones128)
    dis, u0 = _tc_prep(degp, x_pad)
    acc0 = _make_agg(KE, 16, True)(u0, src2d, dst2d, zeros128)
    u1cat = _tc_layer1(acc0, u0, dis, W1, b1.reshape(1, -1))
    u1flat = u1cat.reshape(2 * NP, 128)
    acc1 = _make_agg(KC, 32, False)(u1flat, src2d_both, dst2d, zeros128)
    mu, logstd = _tc_layer2(acc1, u1cat, dis,
                            W_mu.reshape(2, 128, 128), b_mu.reshape(1, -1),
                            W_logstd.reshape(2, 128, 128),
                            b_logstd.reshape(1, -1))
    return mu[:N], logstd[:N]


# speedup vs baseline: 26.9564x; 1.0181x over previous
import functools

import jax
import jax.numpy as jnp
from jax import lax
from jax.experimental import pallas as pl
from jax.experimental.pallas import tpu as pltpu
from jax.experimental.pallas import tpu_sc as plsc

N = 10000
NP = 10240
E = 320000
EP = 327680
ER = EP // 128
KE = ER // 32
KC = ER // 16
SROWS = NP // 16
BLK = 640
GRID = NP // BLK

_F32 = jnp.float32


def _sc_mesh():
    return plsc.VectorSubcoreMesh(
        core_axis_name="c", subcore_axis_name="s", num_cores=2, num_subcores=16)


def _make_deg():
    @functools.partial(
        pl.kernel,
        out_type=jax.ShapeDtypeStruct((2, NP, 128), _F32),
        mesh=_sc_mesh(),
        scratch_types=[
            pltpu.VMEM((KE, 128), jnp.int32),
            pltpu.VMEM((128, 128), _F32),
            pltpu.VMEM_SHARED((NP, 128), _F32),
        ],
    )
    def deg_kernel(dsti, zeros, ones, out, dstv, buf, acc):
        c = lax.axis_index("c")
        s = lax.axis_index("s")
        row0 = (c * 16 + s) * KE
        pltpu.sync_copy(zeros, buf)
        for i in range(SROWS // 128):
            pltpu.sync_copy(buf, acc.at[pl.ds(s * SROWS + i * 128, 128)])
        plsc.subcore_barrier()
        pltpu.sync_copy(dsti.at[pl.ds(row0, KE)], dstv)
        pltpu.sync_copy(ones, buf)

        def body(j, carry):
            pltpu.sync_copy(buf, acc.at[dstv.at[j]], add=True)
            return carry

        lax.fori_loop(0, KE, body, 0)
        plsc.subcore_barrier()
        pltpu.sync_copy(acc.at[pl.ds(s * SROWS, SROWS)],
                        out.at[c, pl.ds(s * SROWS, SROWS)])

    return deg_kernel


def _make_agg(K, RB, edge_split):
    assert K % RB == 0 and RB % 2 == 0
    CH = K // RB

    @functools.partial(
        pl.kernel,
        out_type=jax.ShapeDtypeStruct((2, NP, 128), _F32),
        mesh=_sc_mesh(),
        scratch_types=[
            pltpu.VMEM((RB, 128), jnp.int32),
            pltpu.VMEM((RB, 128), jnp.int32),
            pltpu.VMEM((128, 128), _F32),
            pltpu.VMEM((128, 128), _F32),
            pltpu.VMEM_SHARED((NP, 128), _F32),
            pltpu.SemaphoreType.DMA,
            pltpu.SemaphoreType.DMA,
        ],
    )
    def agg_kernel(table, srci, dsti, zeros, out, srcv, dstv, buf0, buf1,
                   acc, gs0, gs1):
        c = lax.axis_index("c")
        s = lax.axis_index("s")
        row0 = (c * 16 + s) * K if edge_split else s * K
        pltpu.sync_copy(zeros, buf0)
        for i in range(SROWS // 128):
            pltpu.sync_copy(buf0, acc.at[pl.ds(s * SROWS + i * 128, 128)])
        plsc.subcore_barrier()

        bufs = (buf0, buf1)
        sems = (gs0, gs1)

        def chunk(ci, carry):
            base = row0 + ci * RB
            if edge_split:
                pltpu.sync_copy(srci.at[pl.ds(base, RB)], srcv)
            else:
                pltpu.sync_copy(srci.at[c, pl.ds(base, RB)], srcv)
            pltpu.sync_copy(dsti.at[pl.ds(base, RB)], dstv)
            pltpu.async_copy(table.at[srcv.at[0]], buf0, gs0)

            def inner(g, carry2):
                for b in (0, 1):
                    j = 2 * g + b
                    nb = 1 - b

                    @pl.when(j + 1 < RB)
                    def _():
                        pltpu.async_copy(table.at[srcv.at[j + 1]], bufs[nb],
                                         sems[nb])

                    pltpu.make_async_copy(table.at[srcv.at[j]], bufs[b],
                                          sems[b]).wait()
                    pltpu.sync_copy(bufs[b], acc.at[dstv.at[j]], add=True)
                return carry2

            lax.fori_loop(0, RB // 2, inner, 0)
            return carry

        lax.fori_loop(0, CH, chunk, 0)
        plsc.subcore_barrier()
        pltpu.sync_copy(acc.at[pl.ds(s * SROWS, SROWS)],
                        out.at[c, pl.ds(s * SROWS, SROWS)])

    return agg_kernel


def _dot(a, b):
    return jnp.dot(a, b, precision=lax.Precision.DEFAULT,
                   preferred_element_type=_F32)


def _tc_prep(degp, x_pad):
    def body(deg_ref, x_ref, dis_ref, u0_ref):
        d = deg_ref[0][:, :1] + deg_ref[1][:, :1] + 1.0
        dis = lax.rsqrt(jnp.maximum(d, 1.0))
        dis_ref[...] = dis
        u0_ref[...] = dis * x_ref[...]

    return pl.pallas_call(
        body,
        grid=(GRID,),
        in_specs=[
            pl.BlockSpec((2, BLK, 128), lambda i: (0, i, 0)),
            pl.BlockSpec((BLK, 128), lambda i: (i, 0)),
        ],
        out_specs=[
            pl.BlockSpec((BLK, 1), lambda i: (i, 0)),
            pl.BlockSpec((BLK, 128), lambda i: (i, 0)),
        ],
        out_shape=[
            jax.ShapeDtypeStruct((NP, 1), _F32),
            jax.ShapeDtypeStruct((NP, 128), _F32),
        ],
    )(degp, x_pad)


def _tc_layer1(acc0, u0, dis, W1, b1):
    def body(acc_ref, u0_ref, dis_ref, w_ref, b_ref, out_ref):
        i = pl.program_id(0)
        dis = dis_ref[...]
        g0 = dis * (acc_ref[0] + acc_ref[1] + u0_ref[...])
        h = jnp.maximum(_dot(g0, w_ref[...]) + b_ref[...], 0.0)
        u1 = dis * h
        rows = i * BLK + lax.broadcasted_iota(jnp.int32, (BLK, 1), 0)
        u1 = jnp.where(rows < N, u1, 0.0)
        out_ref[0] = u1[:, :128]
        out_ref[1] = u1[:, 128:]

    return pl.pallas_call(
        body,
        grid=(GRID,),
        in_specs=[
            pl.BlockSpec((2, BLK, 128), lambda i: (0, i, 0)),
            pl.BlockSpec((BLK, 128), lambda i: (i, 0)),
            pl.BlockSpec((BLK, 1), lambda i: (i, 0)),
            pl.BlockSpec((128, 256), lambda i: (0, 0)),
            pl.BlockSpec((1, 256), lambda i: (0, 0)),
        ],
        out_specs=[pl.BlockSpec((2, BLK, 128), lambda i: (0, i, 0))],
        out_shape=[jax.ShapeDtypeStruct((2, NP, 128), _F32)],
    )(acc0, u0, dis, W1, b1)[0]


def _tc_layer2(acc1, u1cat, dis, W_mu, b_mu, W_ls, b_ls):
    def body(acc_ref, u1_ref, dis_ref, wmu_ref, bmu_ref, wls_ref, bls_ref,
             mu_ref, ls_ref):
        dis = dis_ref[...]
        g1a = dis * (acc_ref[0] + u1_ref[0])
        g1b = dis * (acc_ref[1] + u1_ref[1])
        mu_ref[...] = _dot(g1a, wmu_ref[0]) + _dot(g1b, wmu_ref[1]) + bmu_ref[...]
        ls_ref[...] = _dot(g1a, wls_ref[0]) + _dot(g1b, wls_ref[1]) + bls_ref[...]

    return pl.pallas_call(
        body,
        grid=(GRID,),
        in_specs=[
            pl.BlockSpec((2, BLK, 128), lambda i: (0, i, 0)),
            pl.BlockSpec((2, BLK, 128), lambda i: (0, i, 0)),
            pl.BlockSpec((BLK, 1), lambda i: (i, 0)),
            pl.BlockSpec((2, 128, 128), lambda i: (0, 0, 0)),
            pl.BlockSpec((1, 128), lambda i: (0, 0)),
            pl.BlockSpec((2, 128, 128), lambda i: (0, 0, 0)),
            pl.BlockSpec((1, 128), lambda i: (0, 0)),
        ],
        out_specs=[
            pl.BlockSpec((BLK, 128), lambda i: (i, 0)),
            pl.BlockSpec((BLK, 128), lambda i: (i, 0)),
        ],
        out_shape=[
            jax.ShapeDtypeStruct((NP, 128), _F32),
            jax.ShapeDtypeStruct((NP, 128), _F32),
        ],
    )(acc1, u1cat, dis, W_mu, b_mu, W_ls, b_ls)


def kernel(x, edge_index, W1, b1, W_mu, b_mu, W_logstd, b_logstd):
    ei = edge_index.astype(jnp.int32)
    pad = N + (jnp.arange(EP - E, dtype=jnp.int32) % (NP - N))
    src2d = jnp.concatenate([ei[0], pad]).reshape(ER, 128)
    dst2d = jnp.concatenate([ei[1], pad]).reshape(ER, 128)
    src2d_both = jnp.stack([src2d, src2d + NP])
    x_pad = jnp.pad(x, ((0, NP - N), (0, 0)))
    zeros128 = jnp.zeros((128, 128), _F32)
    ones128 = jnp.ones((128, 128), _F32)

    degp = _make_deg()(dst2d, zeros128, ones128)
    dis, u0 = _tc_prep(degp, x_pad)
    acc0 = _make_agg(KE, 40, True)(u0, src2d, dst2d, zeros128)
    u1cat = _tc_layer1(acc0, u0, dis, W1, b1.reshape(1, -1))
    u1flat = u1cat.reshape(2 * NP, 128)
    acc1 = _make_agg(KC, 40, False)(u1flat, src2d_both, dst2d, zeros128)
    mu, logstd = _tc_layer2(acc1, u1cat, dis,
                            W_mu.reshape(2, 128, 128), b_mu.reshape(1, -1),
                            W_logstd.reshape(2, 128, 128),
                            b_logstd.reshape(1, -1))
    return mu[:N], logstd[:N]
